# baseline (speedup 1.0000x reference)
import jax
import jax.numpy as jnp
from jax.experimental import pallas as pl
from jax.experimental.pallas import tpu as pltpu

_EPS = 1e-12
_BN = 1024


def _c2_kernel(ct_ref, c2_ref):
    c = ct_ref[...]
    c2_ref[...] = jnp.sum(c * c, axis=0, keepdims=True)


def _make_fused_kernel(nb):
    def _fused(z_ref, ct_ref, ones_ref, c2_ref, q_ref, p_ref, zout_ref,
               qb_ref, f_ref):
        i = pl.program_id(0)

        @pl.when(i < nb)
        def _phase1():
            z = z_ref[...]
            zout_ref[...] = z
            zb = z.astype(jnp.bfloat16)
            z2 = jnp.dot(zb * zb, ones_ref[...],
                         preferred_element_type=jnp.float32)
            dot = jnp.dot(zb, ct_ref[...],
                          preferred_element_type=jnp.float32)
            d2 = (z2 + c2_ref[...]) - 2.0 * dot
            d2c = jnp.maximum(d2, _EPS)
            dist = d2c * jax.lax.rsqrt(d2c)
            num = 1.0 / (1.0 + dist)
            s = jnp.sum(num, axis=1, keepdims=True)
            q = num * (1.0 / s)
            q_ref[...] = q
            qb_ref[pl.ds(pl.multiple_of(i * _BN, _BN), _BN), :] = (
                q.astype(jnp.bfloat16))
            fprev = jnp.where(i == 0, 0.0, f_ref[...])
            f_ref[...] = fprev + jnp.sum(q, axis=0, keepdims=True)

        @pl.when(i >= nb)
        def _phase2():
            j = i - nb
            qb = qb_ref[pl.ds(pl.multiple_of(j * _BN, _BN), _BN), :]
            qf = qb.astype(jnp.float32)
            u = (qf * qf) * (1.0 / f_ref[...])
            s = jnp.sum(u, axis=1, keepdims=True)
            p_ref[...] = u * (1.0 / s)

    return _fused


def kernel(z, centroids):
    n, d = z.shape
    k = centroids.shape[0]
    nb = n // _BN

    ct32 = centroids.T
    ct16 = ct32.astype(jnp.bfloat16)
    ones16 = jnp.ones((d, k), jnp.bfloat16)

    c2 = pl.pallas_call(
        _c2_kernel,
        out_shape=jax.ShapeDtypeStruct((1, k), jnp.float32),
        name="c2",
    )(ct32)

    q, p, z_out = pl.pallas_call(
        _make_fused_kernel(nb),
        grid=(2 * nb,),
        in_specs=[
            pl.BlockSpec((_BN, d), lambda i: (jnp.minimum(i, nb - 1), 0)),
            pl.BlockSpec((d, k), lambda i: (0, 0)),
            pl.BlockSpec((d, k), lambda i: (0, 0)),
            pl.BlockSpec((1, k), lambda i: (0, 0)),
        ],
        out_specs=[
            pl.BlockSpec((_BN, k), lambda i: (jnp.minimum(i, nb - 1), 0)),
            pl.BlockSpec((_BN, k), lambda i: (jnp.maximum(i - nb, 0), 0)),
            pl.BlockSpec((_BN, d), lambda i: (jnp.minimum(i, nb - 1), 0)),
        ],
        out_shape=[
            jax.ShapeDtypeStruct((n, k), jnp.float32),
            jax.ShapeDtypeStruct((n, k), jnp.float32),
            jax.ShapeDtypeStruct((n, d), jnp.float32),
        ],
        scratch_shapes=[
            pltpu.VMEM((n, k), jnp.bfloat16),
            pltpu.VMEM((1, k), jnp.float32),
        ],
        compiler_params=pltpu.CompilerParams(
            dimension_semantics=("arbitrary",),
            vmem_limit_bytes=58 * 1024 * 1024,
        ),
        name="qp_fused",
    )(z, ct16, ones16, c2)

    return (z_out, q, p)


# speedup vs baseline: 1.0470x; 1.0470x over previous
import jax
import jax.numpy as jnp
from jax.experimental import pallas as pl
from jax.experimental.pallas import tpu as pltpu

_EPS = 1e-12
_BN = 1024
_PBN = 2048


def _c2_kernel(ct_ref, c2_ref):
    c = ct_ref[...]
    c2_ref[...] = jnp.sum(c * c, axis=0, keepdims=True)


def _make_fused_kernel(nb):
    def _fused(z_ref, ct_ref, ones_ref, c2_ref, q_ref, p_ref, zout_ref,
               qb_ref, f_ref):
        i = pl.program_id(0)

        @pl.when(i < nb)
        def _phase1():
            z = z_ref[...]
            zout_ref[...] = z
            zb = z.astype(jnp.bfloat16)
            z2 = jnp.dot(zb * zb, ones_ref[...],
                         preferred_element_type=jnp.float32)
            dot = jnp.dot(zb, ct_ref[...],
                          preferred_element_type=jnp.float32)
            d2 = (z2 + c2_ref[...]) - 2.0 * dot
            d2c = jnp.maximum(d2, _EPS)
            dist = d2c * jax.lax.rsqrt(d2c)
            num = 1.0 / (1.0 + dist)
            s = jnp.sum(num, axis=1, keepdims=True)
            q = num * (1.0 / s)
            q_ref[...] = q
            qb_ref[pl.ds(pl.multiple_of(i * _BN, _BN), _BN), :] = (
                q.astype(jnp.bfloat16))
            fprev = jnp.where(i == 0, 0.0, f_ref[...])
            f_ref[...] = fprev + jnp.sum(q, axis=0, keepdims=True)

        @pl.when(i >= nb)
        def _phase2():
            j = i - nb
            qb = qb_ref[pl.ds(pl.multiple_of(j * _PBN, _PBN), _PBN), :]
            qf = qb.astype(jnp.float32)
            u = (qf * qf) * (1.0 / f_ref[...])
            s = jnp.sum(u, axis=1, keepdims=True)
            p_ref[...] = u * (1.0 / s)

    return _fused


def kernel(z, centroids):
    n, d = z.shape
    k = centroids.shape[0]
    nb = n // _BN

    ct32 = centroids.T
    ct16 = ct32.astype(jnp.bfloat16)
    ones16 = jnp.ones((d, k), jnp.bfloat16)

    c2 = pl.pallas_call(
        _c2_kernel,
        out_shape=jax.ShapeDtypeStruct((1, k), jnp.float32),
        name="c2",
    )(ct32)

    nb2 = n // _PBN
    q, p, z_out = pl.pallas_call(
        _make_fused_kernel(nb),
        grid=(nb + nb2,),
        in_specs=[
            pl.BlockSpec((_BN, d), lambda i: (jnp.minimum(i, nb - 1), 0)),
            pl.BlockSpec((d, k), lambda i: (0, 0)),
            pl.BlockSpec((d, k), lambda i: (0, 0)),
            pl.BlockSpec((1, k), lambda i: (0, 0)),
        ],
        out_specs=[
            pl.BlockSpec((_BN, k), lambda i: (jnp.minimum(i, nb - 1), 0)),
            pl.BlockSpec((_PBN, k), lambda i: (jnp.maximum(i - nb, 0), 0)),
            pl.BlockSpec((_BN, d), lambda i: (jnp.minimum(i, nb - 1), 0)),
        ],
        out_shape=[
            jax.ShapeDtypeStruct((n, k), jnp.float32),
            jax.ShapeDtypeStruct((n, k), jnp.float32),
            jax.ShapeDtypeStruct((n, d), jnp.float32),
        ],
        scratch_shapes=[
            pltpu.VMEM((n, k), jnp.bfloat16),
            pltpu.VMEM((1, k), jnp.float32),
        ],
        compiler_params=pltpu.CompilerParams(
            dimension_semantics=("arbitrary",),
            vmem_limit_bytes=61_000_000,
        ),
        name="qp_fused",
    )(z, ct16, ones16, c2)

    return (z_out, q, p)


# speedup vs baseline: 1.0734x; 1.0252x over previous
import jax
import jax.numpy as jnp
from jax.experimental import pallas as pl
from jax.experimental.pallas import tpu as pltpu

_EPS = 1e-12
_BN = 1024
_PBN = 2048


def _c2_kernel(ct_ref, c2_ref):
    c = ct_ref[...]
    c2_ref[...] = jnp.sum(c * c, axis=0, keepdims=True)


def _make_fused_kernel(nb):
    def _fused(z_ref, ct_ref, ones_ref, c2_ref, q_ref, p_ref, zout_ref,
               qb_ref, f_ref):
        i = pl.program_id(0)

        @pl.when(i < nb)
        def _phase1():
            z = z_ref[...]
            zout_ref[...] = z
            zb = z.astype(jnp.bfloat16)
            z2 = jnp.dot(zb * zb, ones_ref[...],
                         preferred_element_type=jnp.float32)
            dot = jnp.dot(zb, ct_ref[...],
                          preferred_element_type=jnp.float32)
            d2 = (z2 + c2_ref[...]) - 2.0 * dot
            d2c = jnp.maximum(d2, _EPS)
            dist = d2c * jax.lax.rsqrt(d2c)
            num = 1.0 / (1.0 + dist)
            s = jnp.sum(num, axis=1, keepdims=True)
            q = num * (1.0 / s)
            qb_ref[pl.ds(pl.multiple_of(i * _BN, _BN), _BN), :] = (
                q.astype(jnp.bfloat16))
            fprev = jnp.where(i == 0, 0.0, f_ref[...])
            f_ref[...] = fprev + jnp.sum(q, axis=0, keepdims=True)

        @pl.when(i >= nb)
        def _phase2():
            j = i - nb
            qb = qb_ref[pl.ds(pl.multiple_of(j * _PBN, _PBN), _PBN), :]
            qf = qb.astype(jnp.float32)
            q_ref[...] = qf
            u = (qf * qf) * (1.0 / f_ref[...])
            s = jnp.sum(u, axis=1, keepdims=True)
            p_ref[...] = u * (1.0 / s)

    return _fused


def kernel(z, centroids):
    n, d = z.shape
    k = centroids.shape[0]
    nb = n // _BN

    ct32 = centroids.T
    ct16 = ct32.astype(jnp.bfloat16)
    ones16 = jnp.ones((d, k), jnp.bfloat16)

    c2 = pl.pallas_call(
        _c2_kernel,
        out_shape=jax.ShapeDtypeStruct((1, k), jnp.float32),
        name="c2",
    )(ct32)

    nb2 = n // _PBN
    q, p, z_out = pl.pallas_call(
        _make_fused_kernel(nb),
        grid=(nb + nb2,),
        in_specs=[
            pl.BlockSpec((_BN, d), lambda i: (jnp.minimum(i, nb - 1), 0)),
            pl.BlockSpec((d, k), lambda i: (0, 0)),
            pl.BlockSpec((d, k), lambda i: (0, 0)),
            pl.BlockSpec((1, k), lambda i: (0, 0)),
        ],
        out_specs=[
            pl.BlockSpec((_PBN, k), lambda i: (jnp.maximum(i - nb, 0), 0)),
            pl.BlockSpec((_PBN, k), lambda i: (jnp.maximum(i - nb, 0), 0)),
            pl.BlockSpec((_BN, d), lambda i: (jnp.minimum(i, nb - 1), 0)),
        ],
        out_shape=[
            jax.ShapeDtypeStruct((n, k), jnp.float32),
            jax.ShapeDtypeStruct((n, k), jnp.float32),
            jax.ShapeDtypeStruct((n, d), jnp.float32),
        ],
        scratch_shapes=[
            pltpu.VMEM((n, k), jnp.bfloat16),
            pltpu.VMEM((1, k), jnp.float32),
        ],
        compiler_params=pltpu.CompilerParams(
            dimension_semantics=("arbitrary",),
            vmem_limit_bytes=61_000_000,
        ),
        name="qp_fused",
    )(z, ct16, ones16, c2)

    return (z_out, q, p)
